# baseline (speedup 1.0000x reference)
import functools

import jax
import jax.numpy as jnp
from jax import lax
from jax.experimental import pallas as pl
from jax.experimental.pallas import tpu as pltpu
from jax.experimental.pallas import tpu_sc as plsc

NC = 2
NS = 16
NW = NC * NS

NBUF = 4


@functools.partial(jax.jit, static_argnames=("n_s", "n_bb", "d"))
def _sc_embedding_gather(idx_t, table, *, n_s, n_bb, d):
    v = table.shape[0]
    mesh = plsc.VectorSubcoreMesh(core_axis_name="c", subcore_axis_name="s")
    ndb = d // 8
    bpw = n_bb // NW
    units = bpw * n_s
    UNROLL = 8

    @functools.partial(
        pl.kernel,
        mesh=mesh,
        out_type=jax.ShapeDtypeStruct((n_s, ndb, n_bb, 8, 128), jnp.float32),
        scratch_types=[
            pltpu.VMEM((n_s, bpw * 128), jnp.int32),
            pltpu.VMEM((NBUF, 128, d), jnp.float32),
            pltpu.VMEM((NBUF, ndb, 8, 132), jnp.float32),
            pltpu.SemaphoreType.DMA,
            pltpu.SemaphoreType.DMA,
        ],
        compiler_params=pltpu.CompilerParams(
            use_tc_tiling_on_sc=False,
            needs_layout_passes=False,
            disable_bounds_checks=True,
            skip_device_barrier=True,
        ),
    )
    def k(idx_hbm, table_hbm, out_hbm, idx_v, rows_v, tbuf_v, gsem, osem):
        wid = lax.axis_index("s") * NC + lax.axis_index("c")
        pltpu.sync_copy(idx_hbm.at[:, pl.ds(wid * bpw * 128, bpw * 128)], idx_v)

        lane = lax.iota(jnp.int32, 16)
        dhi = [(lane + 16 * j) >> 3 for j in range(d // 16)]
        dmid = [(lane + 16 * j) & 7 for j in range(d // 16)]

        def gather_start(u, buf):
            blk = u // n_s
            s = u % n_s
            idxref = idx_v.at[s, pl.ds(blk * 128, 128)]
            return pltpu.async_copy(table_hbm.at[idxref], rows_v.at[buf], gsem)

        def transpose_unit(buf):
            @plsc.parallel_loop(0, 128, unroll=UNROLL)
            def body(b):
                ibi = jnp.full((16,), b, jnp.int32)
                for j in range(d // 16):
                    val = rows_v[buf, b, pl.ds(16 * j, 16)]
                    plsc.store_scatter(
                        tbuf_v.at[buf], [dhi[j], dmid[j], ibi], val
                    )

        def write_start(u, buf):
            blk = u // n_s
            s = u % n_s
            bb = wid * bpw + blk
            return [
                pltpu.async_copy(
                    tbuf_v.at[buf, db, :, pl.ds(0, 128)],
                    out_hbm.at[s, db, bb, :, :],
                    osem,
                )
                for db in range(ndb)
            ]

        def step(t, _):
            gathers = [gather_start(t * NBUF + b, b) for b in range(NBUF)]
            writes = []
            for b in range(NBUF):
                gathers[b].wait()
                transpose_unit(b)
                writes.extend(write_start(t * NBUF + b, b))
            for w in writes:
                w.wait()
            return _

        lax.fori_loop(0, units // NBUF, step, 0)

    return k(idx_t, table)


def kernel(inputs, table):
    b0, s = inputs.shape
    v, d = table.shape
    idx_t = inputs.T.astype(jnp.int32)
    out5 = _sc_embedding_gather(idx_t, table, n_s=s, n_bb=b0 // 128, d=d)
    return out5.transpose((2, 4, 0, 1, 3)).reshape(b0, s, d)


# speedup vs baseline: 1.0143x; 1.0143x over previous
import functools

import jax
import jax.numpy as jnp
from jax import lax
from jax.experimental import pallas as pl
from jax.experimental.pallas import tpu as pltpu
from jax.experimental.pallas import tpu_sc as plsc

NC = 2
NS = 16
NW = NC * NS

NBUF = 5


@functools.partial(jax.jit, static_argnames=("n_s", "n_bb", "d"))
def _sc_embedding_gather(idx_t, table, *, n_s, n_bb, d):
    v = table.shape[0]
    mesh = plsc.VectorSubcoreMesh(core_axis_name="c", subcore_axis_name="s")
    ndb = d // 8
    bpw = n_bb // NW
    units = bpw * n_s
    UNROLL = 8

    @functools.partial(
        pl.kernel,
        mesh=mesh,
        out_type=jax.ShapeDtypeStruct((n_s, ndb, n_bb, 8, 128), jnp.float32),
        scratch_types=[
            pltpu.VMEM((n_s, bpw * 128), jnp.int32),
            pltpu.VMEM((NBUF, 128, d), jnp.float32),
            pltpu.VMEM((NBUF, ndb, 8, 132), jnp.float32),
            pltpu.SemaphoreType.DMA,
            pltpu.SemaphoreType.DMA,
        ],
        compiler_params=pltpu.CompilerParams(
            use_tc_tiling_on_sc=False,
            needs_layout_passes=False,
            disable_bounds_checks=True,
        ),
    )
    def k(idx_hbm, table_hbm, out_hbm, idx_v, rows_v, tbuf_v, gsem, osem):
        wid = lax.axis_index("s") * NC + lax.axis_index("c")
        pltpu.sync_copy(idx_hbm.at[:, pl.ds(wid * bpw * 128, bpw * 128)], idx_v)

        lane = lax.iota(jnp.int32, 16)
        dhi = [(lane + 16 * j) >> 3 for j in range(d // 16)]
        dmid = [(lane + 16 * j) & 7 for j in range(d // 16)]

        def gather_start(u, buf):
            blk = u // n_s
            s = u % n_s
            idxref = idx_v.at[s, pl.ds(blk * 128, 128)]
            return pltpu.async_copy(table_hbm.at[idxref], rows_v.at[buf], gsem)

        def transpose_unit(buf):
            @plsc.parallel_loop(0, 128, unroll=UNROLL)
            def body(b):
                ibi = jnp.full((16,), b, jnp.int32)
                for j in range(d // 16):
                    val = rows_v[buf, b, pl.ds(16 * j, 16)]
                    plsc.store_scatter(
                        tbuf_v.at[buf], [dhi[j], dmid[j], ibi], val
                    )

        def write_start(u, buf):
            blk = u // n_s
            s = u % n_s
            bb = wid * bpw + blk
            return [
                pltpu.async_copy(
                    tbuf_v.at[buf, db, :, pl.ds(0, 128)],
                    out_hbm.at[s, db, bb, :, :],
                    osem,
                )
                for db in range(ndb)
            ]

        def step(t, _):
            gathers = [gather_start(t * NBUF + b, b) for b in range(NBUF)]
            writes = []
            for b in range(NBUF):
                gathers[b].wait()
                transpose_unit(b)
                writes.extend(write_start(t * NBUF + b, b))
            for w in writes:
                w.wait()
            return _

        lax.fori_loop(0, units // NBUF, step, 0)

    return k(idx_t, table)


def kernel(inputs, table):
    b0, s = inputs.shape
    v, d = table.shape
    idx_t = inputs.T.astype(jnp.int32)
    out5 = _sc_embedding_gather(idx_t, table, n_s=s, n_bb=b0 // 128, d=d)
    return out5.transpose((2, 4, 0, 1, 3)).reshape(b0, s, d)
